# baseline (speedup 1.0000x reference)
import functools

import jax
import jax.numpy as jnp
from jax import lax
from jax.experimental import pallas as pl
from jax.experimental.pallas import tpu as pltpu
from jax.experimental.pallas import tpu_sc as plsc

N_NODES = 10000
D = 128
NC = 2
NS = 16
CHUNK = 128
W0 = 132
W1 = 28
E_PAD = (W0 + W1) * NS * CHUNK
E_ALLOC = E_PAD + 3 * CHUNK
N_PAD = 10240
ZERO_ROWS = N_PAD // NS

_IDX_BYTES = CHUNK * 4
_ROW_BYTES = CHUNK * D * 4


def _sc_scatter(x, src, dst, zrows, zdeg):
    mesh = plsc.VectorSubcoreMesh(core_axis_name="c", subcore_axis_name="s")

    @functools.partial(
        pl.kernel,
        out_type=[
            jax.ShapeDtypeStruct((NC * N_PAD, D), jnp.float32),
            jax.ShapeDtypeStruct((NC * N_PAD,), jnp.float32),
        ],
        mesh=mesh,
        scratch_types=[
            pltpu.VMEM((2, CHUNK), jnp.int32),
            pltpu.VMEM((2, CHUNK), jnp.int32),
            pltpu.VMEM((2, CHUNK), jnp.int32),
            pltpu.VMEM((2, CHUNK), jnp.int32),
            pltpu.VMEM((CHUNK, D), jnp.float32),
            pltpu.VMEM((CHUNK, D), jnp.float32),
            pltpu.VMEM((CHUNK,), jnp.float32),
            pltpu.VMEM_SHARED((N_PAD, D), jnp.float32),
            pltpu.VMEM_SHARED((N_PAD,), jnp.float32),
            pltpu.SemaphoreType.DMA,
            pltpu.SemaphoreType.DMA,
            pltpu.SemaphoreType.DMA,
        ],
    )
    def k(x_hbm, src_hbm, dst_hbm, zr_hbm, zd_hbm, agg_out, deg_out,
          idx0, idx1, idx2, idx3, rows0, rows1, ones, agg_sh, deg_sh,
          gsem, isem, ssem):
        c = lax.axis_index("c")
        s = lax.axis_index("s")
        n_my = lax.select(c == 0, W0, W1)
        base = (s * (W0 + W1) + c * W0) * CHUNK
        idxs = (idx0, idx1, idx2, idx3)
        rows = (rows0, rows1)

        pltpu.sync_copy(zr_hbm.at[pl.ds(s * ZERO_ROWS, ZERO_ROWS)],
                        agg_sh.at[pl.ds(s * ZERO_ROWS, ZERO_ROWS)])

        @pl.when(s == 0)
        def _():
            pltpu.sync_copy(zd_hbm, deg_sh)

        for j in range(CHUNK // 16):
            ones[pl.ds(j * 16, 16)] = jnp.ones((16,), jnp.float32)

        plsc.subcore_barrier()

        def idx_load(g, k_, sync=False):
            off = base + g * CHUNK
            if sync:
                pltpu.sync_copy(src_hbm.at[pl.ds(off, CHUNK)], idxs[k_].at[0])
                pltpu.sync_copy(dst_hbm.at[pl.ds(off, CHUNK)], idxs[k_].at[1])
            else:
                pltpu.async_copy(src_hbm.at[pl.ds(off, CHUNK)],
                                 idxs[k_].at[0], isem)
                pltpu.async_copy(dst_hbm.at[pl.ds(off, CHUNK)],
                                 idxs[k_].at[1], isem)

        def idx_wait(k_):
            pltpu.make_async_copy(src_hbm.at[pl.ds(0, CHUNK)],
                                  idxs[k_].at[0], isem).wait()
            pltpu.make_async_copy(dst_hbm.at[pl.ds(0, CHUNK)],
                                  idxs[k_].at[1], isem).wait()

        def gather(k_, j_):
            pltpu.async_copy(x_hbm.at[idxs[k_].at[0]], rows[j_], gsem)

        def gather_wait(k_, j_):
            pltpu.make_async_copy(x_hbm.at[idxs[k_].at[0]], rows[j_],
                                  gsem).wait()

        def scatters(k_, j_):
            pltpu.async_copy(rows[j_], agg_sh.at[idxs[k_].at[1]], ssem,
                             add=True)
            pltpu.async_copy(ones, deg_sh.at[idxs[k_].at[1]], ssem, add=True)

        def scatters_wait(k_, j_):
            pltpu.make_async_copy(rows[j_], agg_sh.at[idxs[k_].at[1]],
                                  ssem).wait()
            pltpu.make_async_copy(ones, deg_sh.at[idxs[k_].at[1]],
                                  ssem).wait()

        idx_load(0, 0, sync=True)
        idx_load(1, 1)
        idx_load(2, 2)
        gather(0, 0)

        def quad(q, carry):
            for j in range(4):
                g = 4 * q + j
                kj = j
                rj = j % 2
                gather_wait(kj, rj)

                @pl.when(g >= 1)
                def _():
                    scatters_wait((j - 1) % 4, (j - 1) % 2)

                @pl.when(g + 1 < n_my)
                def _():
                    idx_wait((j + 1) % 4)
                    gather((j + 1) % 4, (j + 1) % 2)

                scatters(kj, rj)
                idx_load(g + 3, (j + 3) % 4)
            return carry

        lax.fori_loop(0, n_my // 4, quad, 0)

        scatters_wait(3, 1)
        for _ in range(3):
            idx_wait(0)

        plsc.subcore_barrier()

        pltpu.sync_copy(agg_sh.at[pl.ds(s * ZERO_ROWS, ZERO_ROWS)],
                        agg_out.at[pl.ds(c * N_PAD + s * ZERO_ROWS, ZERO_ROWS)])

        @pl.when(s == 0)
        def _():
            pltpu.sync_copy(deg_sh, deg_out.at[pl.ds(c * N_PAD, N_PAD)])

    return k(x, src, dst, zrows, zdeg)


def _tc_finish(agg, deg, W, b):
    blk = 1024
    grid = (N_PAD // blk,)

    def body(a_ref, d_ref, w_ref, b_ref, o_ref):
        a = a_ref[0] + a_ref[1]
        dg = d_ref[0] + d_ref[1]
        o_ref[...] = (jnp.dot(a, w_ref[...], preferred_element_type=jnp.float32)
                      + dg[:, None] * b_ref[...])

    return pl.pallas_call(
        body,
        grid=grid,
        in_specs=[
            pl.BlockSpec((NC, blk, D), lambda i: (0, i, 0)),
            pl.BlockSpec((NC, blk), lambda i: (0, i)),
            pl.BlockSpec((D, D), lambda i: (0, 0)),
            pl.BlockSpec((1, D), lambda i: (0, 0)),
        ],
        out_specs=pl.BlockSpec((blk, D), lambda i: (i, 0)),
        out_shape=jax.ShapeDtypeStruct((N_PAD, D), jnp.float32),
    )(agg, deg, W, b.reshape(1, D))


def kernel(x, edge_index, W, b):
    e = edge_index.astype(jnp.int32)
    n_edges = e.shape[1]
    pad = E_ALLOC - n_edges
    pad_dst = N_NODES + (jnp.arange(pad, dtype=jnp.int32) % (N_PAD - N_NODES))
    src = jnp.concatenate([e[0], jnp.zeros((pad,), jnp.int32)])
    dst = jnp.concatenate([e[1], pad_dst])
    zrows = jnp.zeros((N_PAD, D), jnp.float32)
    zdeg = jnp.zeros((N_PAD,), jnp.float32)
    agg, deg = _sc_scatter(x, src, dst, zrows, zdeg)
    out = _tc_finish(agg.reshape(NC, N_PAD, D), deg.reshape(NC, N_PAD), W, b)
    return out[:N_NODES]


# speedup vs baseline: 1.5165x; 1.5165x over previous
import functools

import jax
import jax.numpy as jnp
from jax import lax
from jax.experimental import pallas as pl
from jax.experimental.pallas import tpu as pltpu
from jax.experimental.pallas import tpu_sc as plsc

N_NODES = 10000
D = 128
NC = 2
NS = 16
CHUNK = 128
N_EDGES = 320000
TOT_CHUNKS = N_EDGES // CHUNK
W0 = 99
W0_EXTRA = 4
SC0_TOTAL = W0 * NS + W0_EXTRA
W1 = (TOT_CHUNKS - SC0_TOTAL) // NS
N_PAD = 10240
ZERO_ROWS = N_PAD // NS


def _sc_scatter(x, e, zdeg):
    mesh = plsc.VectorSubcoreMesh(core_axis_name="c", subcore_axis_name="s")

    @functools.partial(
        pl.kernel,
        out_type=[
            jax.ShapeDtypeStruct((NC * N_PAD, D), jnp.float32),
            jax.ShapeDtypeStruct((NC * N_PAD,), jnp.float32),
        ],
        mesh=mesh,
        scratch_types=[
            pltpu.VMEM((2, CHUNK), jnp.int32),
            pltpu.VMEM((CHUNK, D), jnp.float32),
            pltpu.VMEM((CHUNK,), jnp.float32),
            pltpu.VMEM_SHARED((N_PAD, D), jnp.float32),
            pltpu.VMEM_SHARED((N_PAD,), jnp.float32),
            pltpu.SemaphoreType.DMA,
        ],
    )
    def k(x_hbm, e_hbm, zd_hbm, agg_out, deg_out,
          idx, rows, ones, agg_sh, deg_sh, gsem):
        c = lax.axis_index("c")
        s = lax.axis_index("s")
        n_my = lax.select(c == 0,
                          W0 + jnp.where(s < W0_EXTRA, 1, 0),
                          jnp.full((), W1, jnp.int32))
        base = lax.select(c == 0,
                          s * W0 + jnp.minimum(s, W0_EXTRA),
                          SC0_TOTAL + s * W1) * CHUNK

        def zrow(r, carry):
            for j in range(D // 16):
                rows[r, pl.ds(j * 16, 16)] = jnp.zeros((16,), jnp.float32)
            return carry

        lax.fori_loop(0, CHUNK, zrow, 0)
        for t in range(ZERO_ROWS // CHUNK):
            pltpu.sync_copy(rows,
                            agg_sh.at[pl.ds(s * ZERO_ROWS + t * CHUNK, CHUNK)])

        @pl.when(s == 0)
        def _():
            pltpu.sync_copy(zd_hbm, deg_sh)

        for j in range(CHUNK // 16):
            ones[pl.ds(j * 16, 16)] = jnp.ones((16,), jnp.float32)

        plsc.subcore_barrier()

        def body(g, carry):
            off = base + g * CHUNK
            pltpu.sync_copy(e_hbm.at[0, pl.ds(off, CHUNK)], idx.at[0])
            pltpu.sync_copy(e_hbm.at[1, pl.ds(off, CHUNK)], idx.at[1])
            pltpu.async_copy(x_hbm.at[idx.at[0]], rows, gsem).wait()
            pltpu.sync_copy(rows, agg_sh.at[idx.at[1]], add=True)
            pltpu.sync_copy(ones, deg_sh.at[idx.at[1]], add=True)
            return carry

        lax.fori_loop(0, n_my, body, 0)

        plsc.subcore_barrier()

        pltpu.sync_copy(agg_sh.at[pl.ds(s * ZERO_ROWS, ZERO_ROWS)],
                        agg_out.at[pl.ds(c * N_PAD + s * ZERO_ROWS, ZERO_ROWS)])

        @pl.when(s == 0)
        def _():
            pltpu.sync_copy(deg_sh, deg_out.at[pl.ds(c * N_PAD, N_PAD)])

    return k(x, e, zdeg)


def _tc_finish(agg, deg, W, b):
    blk = 1024
    grid = (N_PAD // blk,)

    def body(a_ref, d_ref, w_ref, b_ref, o_ref):
        a = a_ref[0] + a_ref[1]
        dg = d_ref[0] + d_ref[1]
        o_ref[...] = (jnp.dot(a, w_ref[...], preferred_element_type=jnp.float32)
                      + dg[:, None] * b_ref[...])

    return pl.pallas_call(
        body,
        grid=grid,
        in_specs=[
            pl.BlockSpec((NC, blk, D), lambda i: (0, i, 0)),
            pl.BlockSpec((NC, blk), lambda i: (0, i)),
            pl.BlockSpec((D, D), lambda i: (0, 0)),
            pl.BlockSpec((1, D), lambda i: (0, 0)),
        ],
        out_specs=pl.BlockSpec((blk, D), lambda i: (i, 0)),
        out_shape=jax.ShapeDtypeStruct((N_PAD, D), jnp.float32),
    )(agg, deg, W, b.reshape(1, D))


def kernel(x, edge_index, W, b):
    e = edge_index.astype(jnp.int32)
    zdeg = jnp.zeros((N_PAD,), jnp.float32)
    agg, deg = _sc_scatter(x, e, zdeg)
    out = _tc_finish(agg.reshape(NC, N_PAD, D), deg.reshape(NC, N_PAD), W, b)
    return out[:N_NODES]


# speedup vs baseline: 1.8462x; 1.2174x over previous
import functools

import jax
import jax.numpy as jnp
from jax import lax
from jax.experimental import pallas as pl
from jax.experimental.pallas import tpu as pltpu
from jax.experimental.pallas import tpu_sc as plsc

N_NODES = 10000
D = 128
NC = 2
NS = 16
CHUNK = 128
N_EDGES = 320000
TOT_CHUNKS = N_EDGES // CHUNK
WPT = 78
EXTRA = 2
PER_CORE = WPT * NS + EXTRA
N_PAD = 10240
ZERO_ROWS = N_PAD // NS


def _sc_scatter(x, e, zdeg):
    mesh = plsc.VectorSubcoreMesh(core_axis_name="c", subcore_axis_name="s")

    @functools.partial(
        pl.kernel,
        out_type=[
            jax.ShapeDtypeStruct((NC * N_PAD, D), jnp.float32),
            jax.ShapeDtypeStruct((NC * N_PAD,), jnp.float32),
        ],
        mesh=mesh,
        scratch_types=[
            pltpu.VMEM((2, CHUNK), jnp.int32),
            pltpu.VMEM((CHUNK, D), jnp.float32),
            pltpu.VMEM((CHUNK,), jnp.float32),
            pltpu.VMEM_SHARED((N_PAD, D), jnp.float32),
            pltpu.VMEM_SHARED((N_PAD,), jnp.float32),
            pltpu.SemaphoreType.DMA,
        ],
    )
    def k(x_hbm, e_hbm, zd_hbm, agg_out, deg_out,
          idx, rows, ones, agg_sh, deg_sh, gsem):
        c = lax.axis_index("c")
        s = lax.axis_index("s")
        n_my = WPT + jnp.where(s < EXTRA, 1, 0)
        base = (c * PER_CORE + s * WPT + jnp.minimum(s, EXTRA)) * CHUNK

        def zrow(r, carry):
            for j in range(D // 16):
                rows[r, pl.ds(j * 16, 16)] = jnp.zeros((16,), jnp.float32)
            return carry

        lax.fori_loop(0, CHUNK, zrow, 0)
        for t in range(ZERO_ROWS // CHUNK):
            pltpu.sync_copy(rows,
                            agg_sh.at[pl.ds(s * ZERO_ROWS + t * CHUNK, CHUNK)])

        @pl.when(s == 0)
        def _():
            pltpu.sync_copy(zd_hbm, deg_sh)

        for j in range(CHUNK // 16):
            ones[pl.ds(j * 16, 16)] = jnp.ones((16,), jnp.float32)

        plsc.subcore_barrier()

        def body(g, carry):
            off = base + g * CHUNK
            pltpu.sync_copy(e_hbm.at[0, pl.ds(off, CHUNK)], idx.at[0])
            pltpu.sync_copy(e_hbm.at[1, pl.ds(off, CHUNK)], idx.at[1])
            pltpu.async_copy(x_hbm.at[idx.at[0]], rows, gsem).wait()
            pltpu.sync_copy(rows, agg_sh.at[idx.at[1]], add=True)
            pltpu.sync_copy(ones, deg_sh.at[idx.at[1]], add=True)
            return carry

        lax.fori_loop(0, n_my, body, 0)

        plsc.subcore_barrier()

        pltpu.sync_copy(agg_sh.at[pl.ds(s * ZERO_ROWS, ZERO_ROWS)],
                        agg_out.at[pl.ds(c * N_PAD + s * ZERO_ROWS, ZERO_ROWS)])

        @pl.when(s == 0)
        def _():
            pltpu.sync_copy(deg_sh, deg_out.at[pl.ds(c * N_PAD, N_PAD)])

    return k(x, e, zdeg)


def _tc_finish(agg, deg, W, b):
    blk = 1024
    grid = (N_PAD // blk,)

    def body(a_ref, d_ref, w_ref, b_ref, o_ref):
        a = a_ref[0] + a_ref[1]
        dg = d_ref[0] + d_ref[1]
        o_ref[...] = (jnp.dot(a, w_ref[...], preferred_element_type=jnp.float32)
                      + dg[:, None] * b_ref[...])

    return pl.pallas_call(
        body,
        grid=grid,
        in_specs=[
            pl.BlockSpec((NC, blk, D), lambda i: (0, i, 0)),
            pl.BlockSpec((NC, blk), lambda i: (0, i)),
            pl.BlockSpec((D, D), lambda i: (0, 0)),
            pl.BlockSpec((1, D), lambda i: (0, 0)),
        ],
        out_specs=pl.BlockSpec((blk, D), lambda i: (i, 0)),
        out_shape=jax.ShapeDtypeStruct((N_PAD, D), jnp.float32),
    )(agg, deg, W, b.reshape(1, D))


def kernel(x, edge_index, W, b):
    e = edge_index.astype(jnp.int32)
    zdeg = jnp.zeros((N_PAD,), jnp.float32)
    agg, deg = _sc_scatter(x, e, zdeg)
    out = _tc_finish(agg.reshape(NC, N_PAD, D), deg.reshape(NC, N_PAD), W, b)
    return out[:N_NODES]


# speedup vs baseline: 3.2187x; 1.7435x over previous
import functools

import jax
import jax.numpy as jnp
from jax import lax
from jax.experimental import pallas as pl
from jax.experimental.pallas import tpu as pltpu
from jax.experimental.pallas import tpu_sc as plsc

N_NODES = 10000
D = 128
NC = 2
NS = 16
CHUNK = 128
N_EDGES = 320000
TOT_CHUNKS = N_EDGES // CHUNK
WPT = 76
BIG_TILES = 17
N_PAD = 10240
ZERO_ROWS = N_PAD // NS


def _sc_scatter(x, e, zdeg):
    mesh = plsc.VectorSubcoreMesh(core_axis_name="c", subcore_axis_name="s")

    @functools.partial(
        pl.kernel,
        out_type=[
            jax.ShapeDtypeStruct((NC * N_PAD, D), jnp.float32),
            jax.ShapeDtypeStruct((NC * N_PAD,), jnp.float32),
        ],
        mesh=mesh,
        scratch_types=[
            pltpu.VMEM((2, CHUNK), jnp.int32),
            pltpu.VMEM((2, CHUNK), jnp.int32),
            pltpu.VMEM((2, CHUNK), jnp.int32),
            pltpu.VMEM((2, CHUNK), jnp.int32),
            pltpu.VMEM((CHUNK, D), jnp.float32),
            pltpu.VMEM((CHUNK, D), jnp.float32),
            pltpu.VMEM((CHUNK,), jnp.float32),
            pltpu.VMEM_SHARED((N_PAD, D), jnp.float32),
            pltpu.VMEM_SHARED((N_PAD,), jnp.float32),
            pltpu.SemaphoreType.DMA,
            pltpu.SemaphoreType.DMA,
            pltpu.SemaphoreType.DMA,
        ],
    )
    def k(x_hbm, e_hbm, zd_hbm, agg_out, deg_out,
          idx0, idx1, idx2, idx3, rows_a, rows_b, ones, agg_sh, deg_sh,
          gsem, isem, ssem):
        c = lax.axis_index("c")
        s = lax.axis_index("s")
        flat = c * NS + s
        n_my = WPT + jnp.where(flat < BIG_TILES, 4, 0)
        base = (flat * WPT + 4 * jnp.minimum(flat, BIG_TILES)) * CHUNK
        idxs = (idx0, idx1, idx2, idx3)
        rows = (rows_a, rows_b)

        def zrow(r, carry):
            for j in range(D // 16):
                rows_a[r, pl.ds(j * 16, 16)] = jnp.zeros((16,), jnp.float32)
            return carry

        lax.fori_loop(0, CHUNK, zrow, 0)
        for t in range(ZERO_ROWS // CHUNK):
            pltpu.sync_copy(rows_a,
                            agg_sh.at[pl.ds(s * ZERO_ROWS + t * CHUNK, CHUNK)])

        @pl.when(s == 0)
        def _():
            pltpu.sync_copy(zd_hbm, deg_sh)

        for j in range(CHUNK // 16):
            ones[pl.ds(j * 16, 16)] = jnp.ones((16,), jnp.float32)

        plsc.subcore_barrier()

        def idx_load(g, k_, sync=False):
            off = base + jnp.minimum(g, n_my - 1) * CHUNK
            if sync:
                pltpu.sync_copy(e_hbm.at[0, pl.ds(off, CHUNK)], idxs[k_].at[0])
                pltpu.sync_copy(e_hbm.at[1, pl.ds(off, CHUNK)], idxs[k_].at[1])
            else:
                pltpu.async_copy(e_hbm.at[0, pl.ds(off, CHUNK)],
                                 idxs[k_].at[0], isem)
                pltpu.async_copy(e_hbm.at[1, pl.ds(off, CHUNK)],
                                 idxs[k_].at[1], isem)

        def idx_wait(k_):
            pltpu.make_async_copy(e_hbm.at[0, pl.ds(0, CHUNK)],
                                  idxs[k_].at[0], isem).wait()
            pltpu.make_async_copy(e_hbm.at[1, pl.ds(0, CHUNK)],
                                  idxs[k_].at[1], isem).wait()

        def gather(k_, j_):
            pltpu.async_copy(x_hbm.at[idxs[k_].at[0]], rows[j_], gsem)

        def gather_wait(k_, j_):
            pltpu.make_async_copy(x_hbm.at[idxs[k_].at[0]], rows[j_],
                                  gsem).wait()

        def scatters(k_, j_):
            pltpu.async_copy(rows[j_], agg_sh.at[idxs[k_].at[1]], ssem,
                             add=True)
            pltpu.async_copy(ones, deg_sh.at[idxs[k_].at[1]], ssem, add=True)

        def scatters_wait(k_, j_):
            pltpu.make_async_copy(rows[j_], agg_sh.at[idxs[k_].at[1]],
                                  ssem).wait()
            pltpu.make_async_copy(ones, deg_sh.at[idxs[k_].at[1]],
                                  ssem).wait()

        idx_load(0, 0, sync=True)
        idx_load(1, 1)
        idx_load(2, 2)
        gather(0, 0)

        def quad(q, carry):
            for j in range(4):
                g = 4 * q + j
                gather_wait(j, j % 2)

                @pl.when(g >= 1)
                def _():
                    scatters_wait((j - 1) % 4, (j - 1) % 2)

                @pl.when(g + 1 < n_my)
                def _():
                    idx_wait((j + 1) % 4)
                    gather((j + 1) % 4, (j + 1) % 2)

                scatters(j, j % 2)
                idx_load(g + 3, (j + 3) % 4)
            return carry

        lax.fori_loop(0, n_my // 4, quad, 0)

        scatters_wait(3, 1)
        for _ in range(3):
            idx_wait(0)

        plsc.subcore_barrier()

        pltpu.sync_copy(agg_sh.at[pl.ds(s * ZERO_ROWS, ZERO_ROWS)],
                        agg_out.at[pl.ds(c * N_PAD + s * ZERO_ROWS, ZERO_ROWS)])

        @pl.when(s == 0)
        def _():
            pltpu.sync_copy(deg_sh, deg_out.at[pl.ds(c * N_PAD, N_PAD)])

    return k(x, e, zdeg)


def _tc_finish(agg, deg, W, b):
    blk = 1024
    grid = (N_PAD // blk,)

    def body(a_ref, d_ref, w_ref, b_ref, o_ref):
        a = a_ref[0] + a_ref[1]
        dg = d_ref[0] + d_ref[1]
        o_ref[...] = (jnp.dot(a, w_ref[...], preferred_element_type=jnp.float32)
                      + dg[:, None] * b_ref[...])

    return pl.pallas_call(
        body,
        grid=grid,
        in_specs=[
            pl.BlockSpec((NC, blk, D), lambda i: (0, i, 0)),
            pl.BlockSpec((NC, blk), lambda i: (0, i)),
            pl.BlockSpec((D, D), lambda i: (0, 0)),
            pl.BlockSpec((1, D), lambda i: (0, 0)),
        ],
        out_specs=pl.BlockSpec((blk, D), lambda i: (i, 0)),
        out_shape=jax.ShapeDtypeStruct((N_PAD, D), jnp.float32),
    )(agg, deg, W, b.reshape(1, D))


def kernel(x, edge_index, W, b):
    e = edge_index.astype(jnp.int32)
    zdeg = jnp.zeros((N_PAD,), jnp.float32)
    agg, deg = _sc_scatter(x, e, zdeg)
    out = _tc_finish(agg.reshape(NC, N_PAD, D), deg.reshape(NC, N_PAD), W, b)
    return out[:N_NODES]
